# baseline (speedup 1.0000x reference)
import functools

import jax
import jax.numpy as jnp
from jax import lax
from jax.experimental import pallas as pl
from jax.experimental.pallas import tpu as pltpu
from jax.experimental.pallas import tpu_sc as plsc

B = 4096
D = 32
N_SESS = 24
N_HIST = 2
L = 50
LP = 64
GPB = LP // 4
TGT_G = 14
ATT_HID = 36
AH4 = 4 * ATT_HID
H1, H2 = 256, 128

NC, NS = 2, 16
NW = NC * NS

TB = 256
NB = B // TB

N_MAIN = B * N_HIST * LP
N_SESS_IDX = B * N_SESS


def _sc_gather_all(main_idx, sess_idx, hist_tab, sess_tab):
    mp, sp = N_MAIN // NW, N_SESS_IDX // NW
    CH = 1024
    f32 = jnp.float32
    mesh = plsc.VectorSubcoreMesh(core_axis_name="c", subcore_axis_name="s")

    @functools.partial(
        pl.kernel, mesh=mesh,
        out_type=(jax.ShapeDtypeStruct((N_MAIN, D), f32),
                  jax.ShapeDtypeStruct((N_SESS_IDX, D), f32)),
        scratch_types=[pltpu.VMEM((mp,), jnp.int32),
                       pltpu.VMEM((CH, D), f32),
                       pltpu.VMEM((CH, D), f32),
                       pltpu.SemaphoreType.DMA,
                       pltpu.SemaphoreType.DMA],
        compiler_params=pltpu.CompilerParams(use_tc_tiling_on_sc=False),
    )
    def k(midx, sidx, htab, stab, m_out, s_out,
          idx_v, rows0, rows1, gs0, gs1):
        wid = lax.axis_index("s") * NC + lax.axis_index("c")
        bufs = (rows0, rows1)
        gsems = (gs0, gs1)

        def gather_stream(idx_hbm, tab, out, per, chunk):
            base = wid * per
            n = per // chunk
            pltpu.sync_copy(idx_hbm.at[pl.ds(base, per)],
                            idx_v.at[pl.ds(0, per)])

            def g_copy(j):
                return pltpu.make_async_copy(
                    tab.at[idx_v.at[pl.ds(j * chunk, chunk)]],
                    bufs[j % 2], gsems[j % 2])

            g_copy(0).start()
            for j in range(n):
                if j + 1 < n:
                    g_copy(j + 1).start()
                g_copy(j).wait()
                pltpu.sync_copy(bufs[j % 2],
                                out.at[pl.ds(base + j * chunk, chunk)])

        gather_stream(midx, htab, m_out, mp, CH)
        gather_stream(sidx, stab, s_out, sp, CH)

    return k(main_idx, sess_idx, hist_tab, sess_tab)


def _tc_dense(emb_m, emb_s, W4h, W4m, Wtq, ab1q, W2q, E4, F4,
              W1p, W1t0, W1t1, W1s, b1r, al1, W2, b2r, al2, W3, b3r):
    f32 = jnp.float32

    def body(m_ref, s_ref, W4h_ref, W4m_ref, Wtq_ref, ab1q_ref, W2q_ref,
             E4_ref, F4_ref, W1p_ref, W1t0_ref, W1t1_ref, W1s_ref,
             b1_ref, al1_ref, W2_ref, b2_ref, al2_ref, W3_ref, b3_ref,
             out_ref, x1_scr):
        i = pl.program_id(0)
        x1 = jnp.zeros((TB, H1), f32)
        sp3 = s_ref[...].reshape(TB, 6, 128)
        for j in range(6):
            x1 += jnp.dot(sp3[:, j, :], W1s_ref[pl.ds(j * 128, 128), :],
                          preferred_element_type=f32)
        mp3 = m_ref[...].reshape(TB, 2 * GPB, 128)
        for br in range(N_HIST):
            hp3 = mp3[:, br * GPB:(br + 1) * GPB, :]
            hpf = hp3.reshape(TB * GPB, 128)
            tpk = hp3[:, TGT_G, :]
            hmp = (hp3 * tpk[:, None, :]).reshape(TB * GPB, 128)
            a = jnp.dot(hpf, W4h_ref[br], preferred_element_type=f32)
            a += jnp.dot(hmp, W4m_ref[br], preferred_element_type=f32)
            tw4 = jnp.dot(tpk, Wtq_ref[br], preferred_element_type=f32)
            tw4 = tw4 + ab1q_ref[br]
            a3 = a.reshape(TB, GPB, AH4) + tw4[:, None, :]
            a3 = jnp.maximum(a3, 0.0)
            s = jnp.dot(a3.reshape(TB * GPB, AH4), W2q_ref[br],
                        preferred_element_type=f32)
            s3 = s.reshape(TB, GPB, 4)
            lidx = (lax.broadcasted_iota(jnp.int32, (TB, GPB, 4), 1) * 4
                    + lax.broadcasted_iota(jnp.int32, (TB, GPB, 4), 2))
            s3 = jnp.where(lidx < L, s3, -1e30)
            m1 = jnp.max(s3, axis=2, keepdims=True)
            m0 = jnp.max(m1, axis=1, keepdims=True)
            es = jnp.exp(s3 - m0)
            d1 = jnp.sum(es, axis=2, keepdims=True)
            d0 = jnp.sum(d1, axis=1, keepdims=True)
            w3 = es / d0
            wx = jnp.dot(w3.reshape(TB * GPB, 4), E4_ref[...],
                         preferred_element_type=f32)
            pf = jnp.dot(wx * hpf, F4_ref[...],
                         preferred_element_type=f32)
            pooled = jnp.sum(pf.reshape(TB, GPB, D), axis=1)
            x1 += jnp.dot(pooled, W1p_ref[br], preferred_element_type=f32)
            wt = W1t0_ref if br == 0 else W1t1_ref
            x1 += jnp.dot(tpk, wt[...], preferred_element_type=f32)
        x1 += b1_ref[...]
        x1_scr[pl.ds(i * TB, TB), :] = x1

        @pl.when(i == NB - 1)
        def _():
            eps = 1e-8
            x = x1_scr[...]
            m = jnp.sum(x, axis=0, keepdims=True) * (1.0 / B)
            v = jnp.sum((x - m) ** 2, axis=0, keepdims=True) * (1.0 / B)
            ps = jax.nn.sigmoid((x - m) * lax.rsqrt(v + eps))
            x = ps * x + (1.0 - ps) * al1_ref[...] * x
            x = jnp.dot(x, W2_ref[...], preferred_element_type=f32) + b2_ref[...]
            m2 = jnp.sum(x, axis=0, keepdims=True) * (1.0 / B)
            v2 = jnp.sum((x - m2) ** 2, axis=0, keepdims=True) * (1.0 / B)
            ps2 = jax.nn.sigmoid((x - m2) * lax.rsqrt(v2 + eps))
            x = ps2 * x + (1.0 - ps2) * al2_ref[...] * x
            y = jnp.dot(x, W3_ref[...], preferred_element_type=f32) + b3_ref[...]
            out_ref[...] = jax.nn.sigmoid(y)

    const = lambda shape: pl.BlockSpec(shape, lambda i: (0,) * len(shape))
    return pl.pallas_call(
        body,
        grid=(NB,),
        in_specs=[
            pl.BlockSpec((TB * 2 * GPB, 128), lambda i: (i, 0)),
            pl.BlockSpec((TB * 6, 128), lambda i: (i, 0)),
            const((N_HIST, 128, AH4)),
            const((N_HIST, 128, AH4)),
            const((N_HIST, 128, AH4)),
            const((N_HIST, 1, AH4)),
            const((N_HIST, AH4, 4)),
            const((4, 128)),
            const((128, D)),
            const((N_HIST, D, H1)),
            const((128, H1)),
            const((128, H1)),
            const((N_SESS * D, H1)),
            const((1, H1)),
            const((1, H1)),
            const((H1, H2)),
            const((1, H2)),
            const((1, H2)),
            const((H2, 1)),
            const((1, 1)),
        ],
        out_specs=pl.BlockSpec((B, 1), lambda i: (0, 0)),
        out_shape=jax.ShapeDtypeStruct((B, 1), f32),
        scratch_shapes=[pltpu.VMEM((B, H1), f32)],
        compiler_params=pltpu.CompilerParams(
            dimension_semantics=("arbitrary",)),
    )(emb_m, emb_s, W4h, W4m, Wtq, ab1q, W2q, E4, F4,
      W1p, W1t0, W1t1, W1s, b1r, al1, W2, b2r, al2, W3, b3r)


def _blockdiag4(w):
    z = jnp.zeros((N_HIST, D, ATT_HID), w.dtype)
    rows = []
    for g in range(4):
        blocks = [w if k == g else z for k in range(4)]
        rows.append(jnp.concatenate(blocks, axis=2))
    return jnp.concatenate(rows, axis=1)


def kernel(session_ids, history_ids, target_ids, session_table, hist_table,
           att_W1, att_b1, att_W2, att_b2,
           mlp_W1, mlp_b1, alpha1, mlp_W2, mlp_b2, alpha2, mlp_W3, mlp_b3):
    i32 = jnp.int32
    f32 = jnp.float32
    hist_ids = history_ids.astype(i32)
    tgt = target_ids.astype(i32)
    z6 = jnp.zeros((B, N_HIST, 6), i32)
    z4 = jnp.zeros((B, N_HIST, 4), i32)
    tdup = jnp.broadcast_to(tgt[:, :, None], (B, N_HIST, 4))
    main_idx = jnp.concatenate([hist_ids, z6, tdup, z4], axis=2).reshape(-1)
    sess_idx = session_ids.astype(i32).reshape(-1)

    emb_m, emb_s = _sc_gather_all(main_idx, sess_idx, hist_table, session_table)
    emb_m4 = emb_m.reshape(N_MAIN // 4, 128)
    emb_s4 = emb_s.reshape(N_SESS_IDX // 4, 128)

    Wh = att_W1[:, 0:D] + att_W1[:, 2 * D:3 * D]
    Wt = att_W1[:, D:2 * D] - att_W1[:, 2 * D:3 * D]
    Wm = att_W1[:, 3 * D:4 * D]
    W4h = _blockdiag4(Wh)
    W4m = _blockdiag4(Wm)
    Wtq = jnp.concatenate(
        [jnp.concatenate([Wt] * 4, axis=2),
         jnp.zeros((N_HIST, 128 - D, AH4), f32)], axis=1)
    ab1q = jnp.concatenate([att_b1] * 4, axis=1).reshape(N_HIST, 1, AH4)
    w2 = att_W2[:, :, 0]
    eye4 = jnp.eye(4, dtype=f32)
    W2q = jnp.einsum('bh,gk->bghk', w2, eye4).reshape(N_HIST, AH4, 4)
    E4 = (lax.broadcasted_iota(i32, (4, 128), 1) // D
          == lax.broadcasted_iota(i32, (4, 128), 0)).astype(f32)
    F4 = (lax.broadcasted_iota(i32, (128, D), 0) % D
          == lax.broadcasted_iota(i32, (128, D), 1)).astype(f32)

    W1p = jnp.stack([mlp_W1[0:D], mlp_W1[D:2 * D]])
    W1s = mlp_W1[2 * D:2 * D + N_SESS * D]
    W1t0 = jnp.concatenate(
        [mlp_W1[2 * D + N_SESS * D:2 * D + N_SESS * D + D],
         jnp.zeros((128 - D, H1), f32)], axis=0)
    W1t1 = jnp.concatenate(
        [mlp_W1[2 * D + N_SESS * D + D:],
         jnp.zeros((128 - D, H1), f32)], axis=0)
    b1r = mlp_b1.reshape(1, H1)
    al1 = alpha1.reshape(1, H1)
    b2r = mlp_b2.reshape(1, H2)
    al2 = alpha2.reshape(1, H2)
    b3r = mlp_b3.reshape(1, 1)

    out = _tc_dense(emb_m4, emb_s4, W4h, W4m, Wtq, ab1q, W2q, E4, F4,
                    W1p, W1t0, W1t1, W1s, b1r, al1,
                    mlp_W2, b2r, al2, mlp_W3, b3r)
    return out.reshape(B)


# speedup vs baseline: 2.2395x; 2.2395x over previous
import functools

import jax
import jax.numpy as jnp
from jax import lax
from jax.experimental import pallas as pl
from jax.experimental.pallas import tpu as pltpu
from jax.experimental.pallas import tpu_sc as plsc

B = 4096
D = 32
N_SESS = 24
N_HIST = 2
L = 50
LP = 64
GPB = LP // 4
TGT_G = 14
ATT_HID = 36
AH4 = 4 * ATT_HID
H1, H2 = 256, 128

NC, NS = 2, 16
NW = NC * NS

TB = 256
NB = B // TB

N_MAIN = B * N_HIST * LP
N_SESS_IDX = B * N_SESS


def _sc_gather_all(main_idx, sess_idx, hist_tab, sess_tab):
    mp, sp = N_MAIN // NW, N_SESS_IDX // NW
    CH = 1024
    f32 = jnp.float32
    mesh = plsc.VectorSubcoreMesh(core_axis_name="c", subcore_axis_name="s")

    @functools.partial(
        pl.kernel, mesh=mesh,
        out_type=(jax.ShapeDtypeStruct((N_MAIN, D), f32),
                  jax.ShapeDtypeStruct((N_SESS_IDX, D), f32)),
        scratch_types=[pltpu.VMEM((mp,), jnp.int32),
                       pltpu.VMEM((CH, D), f32),
                       pltpu.VMEM((CH, D), f32),
                       pltpu.SemaphoreType.DMA,
                       pltpu.SemaphoreType.DMA],
        compiler_params=pltpu.CompilerParams(use_tc_tiling_on_sc=False),
    )
    def k(midx, sidx, htab, stab, m_out, s_out,
          idx_v, rows0, rows1, gs0, gs1):
        wid = lax.axis_index("s") * NC + lax.axis_index("c")
        bufs = (rows0, rows1)
        gsems = (gs0, gs1)

        def gather_stream(idx_hbm, tab, out, per, chunk):
            base = wid * per
            n = per // chunk
            pltpu.sync_copy(idx_hbm.at[pl.ds(base, per)],
                            idx_v.at[pl.ds(0, per)])

            def g_copy(j):
                return pltpu.make_async_copy(
                    tab.at[idx_v.at[pl.ds(j * chunk, chunk)]],
                    bufs[j % 2], gsems[j % 2])

            g_copy(0).start()
            for j in range(n):
                if j + 1 < n:
                    g_copy(j + 1).start()
                g_copy(j).wait()
                pltpu.sync_copy(bufs[j % 2],
                                out.at[pl.ds(base + j * chunk, chunk)])

        gather_stream(midx, htab, m_out, mp, CH)
        gather_stream(sidx, stab, s_out, sp, CH)

    return k(main_idx, sess_idx, hist_tab, sess_tab)


def _tc_dense(emb_m, emb_s, W4h, W4m, Wtq, ab1q, W2q, E4, F4,
              W1p, W1t0, W1t1, W1s, b1r, al1, W2, b2r, al2, W3, b3r):
    f32 = jnp.float32

    def body(m_ref, s_ref, W4h_ref, W4m_ref, Wtq_ref, ab1q_ref, W2q_ref,
             E4_ref, F4_ref, W1p_ref, W1t0_ref, W1t1_ref, W1s_ref,
             b1_ref, al1_ref, W2_ref, b2_ref, al2_ref, W3_ref, b3_ref,
             out_ref, x1_scr):
        i = pl.program_id(0)
        x1 = jnp.zeros((TB, H1), f32)
        sp3 = s_ref[...].reshape(TB, 6, 128)
        for j in range(6):
            x1 += jnp.dot(sp3[:, j, :], W1s_ref[pl.ds(j * 128, 128), :],
                          preferred_element_type=f32)
        mp3 = m_ref[...].reshape(TB, 2 * GPB, 128)
        for br in range(N_HIST):
            hp3 = mp3[:, br * GPB:(br + 1) * GPB, :]
            hpf = hp3.reshape(TB * GPB, 128)
            tpk = hp3[:, TGT_G, :]
            hmp = (hp3 * tpk[:, None, :]).reshape(TB * GPB, 128)
            a = jnp.dot(hpf, W4h_ref[br], preferred_element_type=f32)
            a += jnp.dot(hmp, W4m_ref[br], preferred_element_type=f32)
            tw4 = jnp.dot(tpk, Wtq_ref[br], preferred_element_type=f32)
            tw4 = tw4 + ab1q_ref[br]
            a3 = a.reshape(TB, GPB, AH4) + tw4[:, None, :]
            a3 = jnp.maximum(a3, 0.0)
            s = jnp.dot(a3.reshape(TB * GPB, AH4), W2q_ref[br],
                        preferred_element_type=f32)
            s3 = s.reshape(TB, GPB, 4)
            lidx = (lax.broadcasted_iota(jnp.int32, (TB, GPB, 4), 1) * 4
                    + lax.broadcasted_iota(jnp.int32, (TB, GPB, 4), 2))
            s3 = jnp.where(lidx < L, s3, -1e30)
            m1 = jnp.max(s3, axis=2, keepdims=True)
            m0 = jnp.max(m1, axis=1, keepdims=True)
            es = jnp.exp(s3 - m0)
            d1 = jnp.sum(es, axis=2, keepdims=True)
            d0 = jnp.sum(d1, axis=1, keepdims=True)
            w3 = es / d0
            wx = jnp.dot(w3.reshape(TB * GPB, 4), E4_ref[...],
                         preferred_element_type=f32)
            pf = jnp.dot(wx * hpf, F4_ref[...],
                         preferred_element_type=f32)
            pooled = jnp.sum(pf.reshape(TB, GPB, D), axis=1)
            x1 += jnp.dot(pooled, W1p_ref[br], preferred_element_type=f32)
            wt = W1t0_ref if br == 0 else W1t1_ref
            x1 += jnp.dot(tpk, wt[...], preferred_element_type=f32)
        x1 += b1_ref[...]
        x1_scr[pl.ds(i * TB, TB), :] = x1

        @pl.when(i == NB - 1)
        def _():
            eps = 1e-8
            x = x1_scr[...]
            m = jnp.sum(x, axis=0, keepdims=True) * (1.0 / B)
            v = jnp.sum((x - m) ** 2, axis=0, keepdims=True) * (1.0 / B)
            ps = jax.nn.sigmoid((x - m) * lax.rsqrt(v + eps))
            x = ps * x + (1.0 - ps) * al1_ref[...] * x
            x = jnp.dot(x, W2_ref[...], preferred_element_type=f32) + b2_ref[...]
            m2 = jnp.sum(x, axis=0, keepdims=True) * (1.0 / B)
            v2 = jnp.sum((x - m2) ** 2, axis=0, keepdims=True) * (1.0 / B)
            ps2 = jax.nn.sigmoid((x - m2) * lax.rsqrt(v2 + eps))
            x = ps2 * x + (1.0 - ps2) * al2_ref[...] * x
            y = jnp.dot(x, W3_ref[...], preferred_element_type=f32) + b3_ref[...]
            out_ref[...] = jax.nn.sigmoid(y)

    const = lambda shape: pl.BlockSpec(shape, lambda i: (0,) * len(shape))
    return pl.pallas_call(
        body,
        grid=(NB,),
        in_specs=[
            pl.BlockSpec((TB * 2 * GPB, 128), lambda i: (i, 0)),
            pl.BlockSpec((TB * 6, 128), lambda i: (i, 0)),
            const((N_HIST, 128, AH4)),
            const((N_HIST, 128, AH4)),
            const((N_HIST, 128, AH4)),
            const((N_HIST, 1, AH4)),
            const((N_HIST, AH4, 4)),
            const((4, 128)),
            const((128, D)),
            const((N_HIST, D, H1)),
            const((128, H1)),
            const((128, H1)),
            const((N_SESS * D, H1)),
            const((1, H1)),
            const((1, H1)),
            const((H1, H2)),
            const((1, H2)),
            const((1, H2)),
            const((H2, 1)),
            const((1, 1)),
        ],
        out_specs=pl.BlockSpec((B, 1), lambda i: (0, 0)),
        out_shape=jax.ShapeDtypeStruct((B, 1), f32),
        scratch_shapes=[pltpu.VMEM((B, H1), f32)],
        compiler_params=pltpu.CompilerParams(
            dimension_semantics=("arbitrary",)),
    )(emb_m, emb_s, W4h, W4m, Wtq, ab1q, W2q, E4, F4,
      W1p, W1t0, W1t1, W1s, b1r, al1, W2, b2r, al2, W3, b3r)


def _blockdiag4(w):
    z = jnp.zeros((N_HIST, D, ATT_HID), w.dtype)
    rows = []
    for g in range(4):
        blocks = [w if k == g else z for k in range(4)]
        rows.append(jnp.concatenate(blocks, axis=2))
    return jnp.concatenate(rows, axis=1)


def kernel(session_ids, history_ids, target_ids, session_table, hist_table,
           att_W1, att_b1, att_W2, att_b2,
           mlp_W1, mlp_b1, alpha1, mlp_W2, mlp_b2, alpha2, mlp_W3, mlp_b3):
    i32 = jnp.int32
    f32 = jnp.float32
    hist_ids = history_ids.astype(i32)
    tgt = target_ids.astype(i32)

    def _pad_ids(width, salt):
        pos = (lax.broadcasted_iota(i32, (B, N_HIST, width), 0) * 2 * width
               + lax.broadcasted_iota(i32, (B, N_HIST, width), 1) * width
               + lax.broadcasted_iota(i32, (B, N_HIST, width), 2) + salt)
        return (pos * 997) % 999999

    z6 = _pad_ids(6, 0)
    z4 = _pad_ids(4, 131071)
    tdup = jnp.broadcast_to(tgt[:, :, None], (B, N_HIST, 4))
    main_idx = jnp.concatenate([hist_ids, z6, tdup, z4], axis=2).reshape(-1)
    sess_idx = session_ids.astype(i32).reshape(-1)

    emb_m, emb_s = _sc_gather_all(main_idx, sess_idx, hist_table, session_table)
    emb_m4 = emb_m.reshape(N_MAIN // 4, 128)
    emb_s4 = emb_s.reshape(N_SESS_IDX // 4, 128)

    Wh = att_W1[:, 0:D] + att_W1[:, 2 * D:3 * D]
    Wt = att_W1[:, D:2 * D] - att_W1[:, 2 * D:3 * D]
    Wm = att_W1[:, 3 * D:4 * D]
    W4h = _blockdiag4(Wh)
    W4m = _blockdiag4(Wm)
    Wtq = jnp.concatenate(
        [jnp.concatenate([Wt] * 4, axis=2),
         jnp.zeros((N_HIST, 128 - D, AH4), f32)], axis=1)
    ab1q = jnp.concatenate([att_b1] * 4, axis=1).reshape(N_HIST, 1, AH4)
    w2 = att_W2[:, :, 0]
    eye4 = jnp.eye(4, dtype=f32)
    W2q = jnp.einsum('bh,gk->bghk', w2, eye4).reshape(N_HIST, AH4, 4)
    E4 = (lax.broadcasted_iota(i32, (4, 128), 1) // D
          == lax.broadcasted_iota(i32, (4, 128), 0)).astype(f32)
    F4 = (lax.broadcasted_iota(i32, (128, D), 0) % D
          == lax.broadcasted_iota(i32, (128, D), 1)).astype(f32)

    W1p = jnp.stack([mlp_W1[0:D], mlp_W1[D:2 * D]])
    W1s = mlp_W1[2 * D:2 * D + N_SESS * D]
    W1t0 = jnp.concatenate(
        [mlp_W1[2 * D + N_SESS * D:2 * D + N_SESS * D + D],
         jnp.zeros((128 - D, H1), f32)], axis=0)
    W1t1 = jnp.concatenate(
        [mlp_W1[2 * D + N_SESS * D + D:],
         jnp.zeros((128 - D, H1), f32)], axis=0)
    b1r = mlp_b1.reshape(1, H1)
    al1 = alpha1.reshape(1, H1)
    b2r = mlp_b2.reshape(1, H2)
    al2 = alpha2.reshape(1, H2)
    b3r = mlp_b3.reshape(1, 1)

    out = _tc_dense(emb_m4, emb_s4, W4h, W4m, Wtq, ab1q, W2q, E4, F4,
                    W1p, W1t0, W1t1, W1s, b1r, al1,
                    mlp_W2, b2r, al2, mlp_W3, b3r)
    return out.reshape(B)
